# baseline (speedup 1.0000x reference)
import jax
import jax.numpy as jnp
from jax import lax
from jax.experimental import pallas as pl
from jax.experimental.pallas import tpu as pltpu
from jax.experimental.pallas import tpu_sc as plsc

N = 10000
D = 128
DH = D // 2
E = 320000
NC = 2
NS = 16
K = 128
NB = 160
EPT = NB * K
E_PAD = EPT * NS
RPT = 640
N_PAD = RPT * NS
ZR = 160
DEGW = 16

_mesh = plsc.VectorSubcoreMesh(core_axis_name="c", subcore_axis_name="s")


def _seg_deg_body(src_hbm, dst_hbm, y_hbm, agg_hbm, deg_hbm,
                  src_v, dst_v, bufs, zbuf, gsems, agg_sh,
                  ones_v, zdeg, deg_sh):
  c = lax.axis_index("c")
  s = lax.axis_index("s")
  wid = c * NS + s
  r0 = s * RPT

  @pl.loop(0, ZR)
  def _(r):
    for j in range(DH // 16):
      zbuf[r, pl.ds(j * 16, 16)] = jnp.zeros((16,), jnp.float32)

  @pl.loop(0, 4)
  def _(i):
    pltpu.sync_copy(zbuf, agg_sh.at[pl.ds(r0 + i * ZR, ZR)])

  @pl.loop(0, ZR)
  def _(r):
    zdeg[r, :] = jnp.zeros((DEGW,), jnp.float32)

  @pl.loop(0, K)
  def _(r):
    ones_v[r, :] = jnp.ones((DEGW,), jnp.float32)

  @pl.loop(0, 4)
  def _(i):
    pltpu.sync_copy(zdeg, deg_sh.at[pl.ds(r0 + i * ZR, ZR)])

  pltpu.sync_copy(src_hbm.at[wid], src_v)
  pltpu.sync_copy(dst_hbm.at[s], dst_v)

  plsc.subcore_barrier()

  pltpu.async_copy(y_hbm.at[src_v.at[0]], bufs[0], gsems[0])

  lo = c * (NB // 2)
  hi = lo + NB // 2

  @pl.loop(0, NB, step=2)
  def _(b):
    pltpu.async_copy(y_hbm.at[src_v.at[b + 1]], bufs[1], gsems[1])
    pltpu.make_async_copy(y_hbm.at[src_v.at[b]], bufs[0], gsems[0]).wait()
    pltpu.sync_copy(bufs[0], agg_sh.at[dst_v.at[b]], add=True)

    @pl.when(jnp.logical_and(b >= lo, b < hi))
    def _():
      pltpu.sync_copy(ones_v, deg_sh.at[dst_v.at[b]], add=True)

    bn = jnp.minimum(b + 2, NB - 1)
    pltpu.async_copy(y_hbm.at[src_v.at[bn]], bufs[0], gsems[0])
    pltpu.make_async_copy(y_hbm.at[src_v.at[b + 1]], bufs[1],
                          gsems[1]).wait()
    pltpu.sync_copy(bufs[1], agg_sh.at[dst_v.at[b + 1]], add=True)

    @pl.when(jnp.logical_and(b + 1 >= lo, b + 1 < hi))
    def _():
      pltpu.sync_copy(ones_v, deg_sh.at[dst_v.at[b + 1]], add=True)

  pltpu.make_async_copy(y_hbm.at[src_v.at[NB - 1]], bufs[0], gsems[0]).wait()

  plsc.subcore_barrier()

  pltpu.sync_copy(agg_sh.at[pl.ds(r0, RPT)],
                  agg_hbm.at[c].at[pl.ds(r0, RPT)])
  pltpu.sync_copy(deg_sh.at[pl.ds(r0, RPT)],
                  deg_hbm.at[c].at[pl.ds(r0, RPT)])


_seg_deg = pl.kernel(
    _seg_deg_body,
    out_type=(
        jax.ShapeDtypeStruct((NC, N_PAD, DH), jnp.float32),
        jax.ShapeDtypeStruct((NC, N_PAD, DEGW), jnp.float32),
    ),
    mesh=_mesh,
    scratch_types=[
        pltpu.VMEM((NB, K), jnp.int32),
        pltpu.VMEM((NB, K), jnp.int32),
        [pltpu.VMEM((K, DH), jnp.float32)] * 2,
        pltpu.VMEM((ZR, DH), jnp.float32),
        [pltpu.SemaphoreType.DMA] * 2,
        pltpu.VMEM_SHARED((N_PAD, DH), jnp.float32),
        pltpu.VMEM((K, DEGW), jnp.float32),
        pltpu.VMEM((ZR, DEGW), jnp.float32),
        pltpu.VMEM_SHARED((N_PAD, DEGW), jnp.float32),
    ],
    compiler_params=pltpu.CompilerParams(use_tc_tiling_on_sc=False),
)


def _seg_body(src_hbm, dst_hbm, y_hbm, agg_hbm,
              src_v, dst_v, bufs, zbuf, gsems, agg_sh):
  c = lax.axis_index("c")
  s = lax.axis_index("s")
  wid = c * NS + s
  r0 = s * RPT

  @pl.loop(0, ZR)
  def _(r):
    for j in range(DH // 16):
      zbuf[r, pl.ds(j * 16, 16)] = jnp.zeros((16,), jnp.float32)

  @pl.loop(0, 4)
  def _(i):
    pltpu.sync_copy(zbuf, agg_sh.at[pl.ds(r0 + i * ZR, ZR)])

  pltpu.sync_copy(src_hbm.at[wid], src_v)
  pltpu.sync_copy(dst_hbm.at[s], dst_v)

  plsc.subcore_barrier()

  pltpu.async_copy(y_hbm.at[src_v.at[0]], bufs[0], gsems[0])

  @pl.loop(0, NB, step=2)
  def _(b):
    pltpu.async_copy(y_hbm.at[src_v.at[b + 1]], bufs[1], gsems[1])
    pltpu.make_async_copy(y_hbm.at[src_v.at[b]], bufs[0], gsems[0]).wait()
    pltpu.sync_copy(bufs[0], agg_sh.at[dst_v.at[b]], add=True)
    bn = jnp.minimum(b + 2, NB - 1)
    pltpu.async_copy(y_hbm.at[src_v.at[bn]], bufs[0], gsems[0])
    pltpu.make_async_copy(y_hbm.at[src_v.at[b + 1]], bufs[1],
                          gsems[1]).wait()
    pltpu.sync_copy(bufs[1], agg_sh.at[dst_v.at[b + 1]], add=True)

  pltpu.make_async_copy(y_hbm.at[src_v.at[NB - 1]], bufs[0], gsems[0]).wait()

  plsc.subcore_barrier()

  pltpu.sync_copy(agg_sh.at[pl.ds(r0, RPT)],
                  agg_hbm.at[c].at[pl.ds(r0, RPT)])


_seg = pl.kernel(
    _seg_body,
    out_type=jax.ShapeDtypeStruct((NC, N_PAD, DH), jnp.float32),
    mesh=_mesh,
    scratch_types=[
        pltpu.VMEM((NB, K), jnp.int32),
        pltpu.VMEM((NB, K), jnp.int32),
        [pltpu.VMEM((K, DH), jnp.float32)] * 2,
        pltpu.VMEM((ZR, DH), jnp.float32),
        [pltpu.SemaphoreType.DMA] * 2,
        pltpu.VMEM_SHARED((N_PAD, DH), jnp.float32),
    ],
    compiler_params=pltpu.CompilerParams(use_tc_tiling_on_sc=False),
)


RB = 1000


def _mm2_body(x_ref, ws_ref, wn_ref, self_ref, y_ref):
  xb = x_ref[...]
  self_ref[...] = jnp.dot(xb, ws_ref[...], preferred_element_type=jnp.float32)
  y = jnp.dot(xb, wn_ref[...], preferred_element_type=jnp.float32)
  y_ref[0] = y[:, :DH]
  y_ref[1] = y[:, DH:]


_mm2 = pl.pallas_call(
    _mm2_body,
    grid=(N // RB,),
    in_specs=[
        pl.BlockSpec((RB, D), lambda i: (i, 0)),
        pl.BlockSpec((D, D), lambda i: (0, 0)),
        pl.BlockSpec((D, D), lambda i: (0, 0)),
    ],
    out_specs=[
        pl.BlockSpec((RB, D), lambda i: (i, 0)),
        pl.BlockSpec((2, RB, DH), lambda i: (0, i, 0)),
    ],
    out_shape=[
        jax.ShapeDtypeStruct((N, D), jnp.float32),
        jax.ShapeDtypeStruct((2, N, DH), jnp.float32),
    ],
)


def _combine_mm_body(s1_ref, a0_ref, a1_ref, d0_ref, d1_ref, b_ref,
                     ws_ref, wn_ref, self2_ref, y2_ref):
  deg = jnp.maximum(d0_ref[:, :1] + d1_ref[:, :1], 1.0)
  agg = jnp.concatenate([a0_ref[...], a1_ref[...]], axis=1)
  h = jnp.maximum(s1_ref[...] + b_ref[...] + agg / deg, 0.0)
  self2_ref[...] = jnp.dot(h, ws_ref[...], preferred_element_type=jnp.float32)
  y2 = jnp.dot(h, wn_ref[...], preferred_element_type=jnp.float32)
  y2_ref[0] = y2[:, :DH]
  y2_ref[1] = y2[:, DH:]


_combine_mm = pl.pallas_call(
    _combine_mm_body,
    grid=(N // RB,),
    in_specs=[
        pl.BlockSpec((RB, D), lambda i: (i, 0)),
        pl.BlockSpec((RB, DH), lambda i: (i, 0)),
        pl.BlockSpec((RB, DH), lambda i: (i, 0)),
        pl.BlockSpec((RB, DEGW), lambda i: (i, 0)),
        pl.BlockSpec((RB, DEGW), lambda i: (i, 0)),
        pl.BlockSpec((1, D), lambda i: (0, 0)),
        pl.BlockSpec((D, D), lambda i: (0, 0)),
        pl.BlockSpec((D, D), lambda i: (0, 0)),
    ],
    out_specs=[
        pl.BlockSpec((RB, D), lambda i: (i, 0)),
        pl.BlockSpec((2, RB, DH), lambda i: (0, i, 0)),
    ],
    out_shape=[
        jax.ShapeDtypeStruct((N, D), jnp.float32),
        jax.ShapeDtypeStruct((2, N, DH), jnp.float32),
    ],
)


def _final_body(s2_ref, a0_ref, a1_ref, d0_ref, d1_ref, b_ref, out_ref):
  deg = jnp.maximum(d0_ref[:, :1] + d1_ref[:, :1], 1.0)
  agg = jnp.concatenate([a0_ref[...], a1_ref[...]], axis=1)
  out_ref[...] = s2_ref[...] + b_ref[...] + agg / deg


_final = pl.pallas_call(
    _final_body,
    grid=(N // RB,),
    in_specs=[
        pl.BlockSpec((RB, D), lambda i: (i, 0)),
        pl.BlockSpec((RB, DH), lambda i: (i, 0)),
        pl.BlockSpec((RB, DH), lambda i: (i, 0)),
        pl.BlockSpec((RB, DEGW), lambda i: (i, 0)),
        pl.BlockSpec((RB, DEGW), lambda i: (i, 0)),
        pl.BlockSpec((1, D), lambda i: (0, 0)),
    ],
    out_specs=pl.BlockSpec((RB, D), lambda i: (i, 0)),
    out_shape=jax.ShapeDtypeStruct((N, D), jnp.float32),
)


@jax.jit
def kernel(edge_index, in_feat, W_self1, W_neigh1, b1, W_self2, W_neigh2, b2):
  src = edge_index[0]
  dst = edge_index[1]
  pad = E_PAD - E
  src_t = jnp.concatenate(
      [src, jnp.zeros((pad,), jnp.int32)]).reshape(NS, NB, K)
  src_r = jnp.concatenate([src_t, src_t + N], axis=0)
  dst_r = jnp.concatenate(
      [dst, jnp.full((pad,), N, jnp.int32)]).reshape(NS, NB, K)

  self1, y1 = _mm2(in_feat, W_self1, W_neigh1)
  agg1, deg = _seg_deg(src_r, dst_r, y1.reshape(2 * N, DH))
  self2, y2 = _combine_mm(self1, agg1[0], agg1[1], deg[0], deg[1],
                          b1.reshape(1, D), W_self2, W_neigh2)
  agg2 = _seg(src_r, dst_r, y2.reshape(2 * N, DH))
  out = _final(self2, agg2[0], agg2[1], deg[0], deg[1], b2.reshape(1, D))
  return out


# speedup vs baseline: 1.0866x; 1.0866x over previous
import jax
import jax.numpy as jnp
from jax import lax
from jax.experimental import pallas as pl
from jax.experimental.pallas import tpu as pltpu
from jax.experimental.pallas import tpu_sc as plsc

N = 10000
D = 128
DH = D // 2
E = 320000
NC = 2
NS = 16
K = 128
NB = 160
EPT = NB * K
E_PAD = EPT * NS
RPT = 640
N_PAD = RPT * NS
ZR = 160
DEGW = 16

_mesh = plsc.VectorSubcoreMesh(core_axis_name="c", subcore_axis_name="s")


def _seg_deg_body(src_hbm, dst_hbm, y_hbm, agg_hbm, deg_hbm,
                  src_v, dst_v, bufs, zbuf, gsems, agg_sh,
                  ones_v, zdeg, deg_sh):
  c = lax.axis_index("c")
  s = lax.axis_index("s")
  wid = c * NS + s
  r0 = s * RPT

  @pl.loop(0, ZR)
  def _(r):
    for j in range(DH // 16):
      zbuf[r, pl.ds(j * 16, 16)] = jnp.zeros((16,), jnp.float32)

  @pl.loop(0, 4)
  def _(i):
    pltpu.sync_copy(zbuf, agg_sh.at[pl.ds(r0 + i * ZR, ZR)])

  @pl.loop(0, ZR)
  def _(r):
    zdeg[r, :] = jnp.zeros((DEGW,), jnp.float32)

  @pl.loop(0, K)
  def _(r):
    ones_v[r, :] = jnp.ones((DEGW,), jnp.float32)

  @pl.loop(0, 4)
  def _(i):
    pltpu.sync_copy(zdeg, deg_sh.at[pl.ds(r0 + i * ZR, ZR)])

  pltpu.sync_copy(src_hbm.at[wid], src_v)
  pltpu.sync_copy(dst_hbm.at[s], dst_v)

  plsc.subcore_barrier()

  pltpu.async_copy(y_hbm.at[src_v.at[0]], bufs[0], gsems[0])

  @pl.loop(0, NB, step=2)
  def _(b):
    pltpu.async_copy(y_hbm.at[src_v.at[b + 1]], bufs[1], gsems[1])
    pltpu.make_async_copy(y_hbm.at[src_v.at[b]], bufs[0], gsems[0]).wait()
    pltpu.sync_copy(bufs[0], agg_sh.at[dst_v.at[b]], add=True)
    pltpu.sync_copy(ones_v, deg_sh.at[dst_v.at[b]], add=True)
    bn = jnp.minimum(b + 2, NB - 1)
    pltpu.async_copy(y_hbm.at[src_v.at[bn]], bufs[0], gsems[0])
    pltpu.make_async_copy(y_hbm.at[src_v.at[b + 1]], bufs[1],
                          gsems[1]).wait()
    pltpu.sync_copy(bufs[1], agg_sh.at[dst_v.at[b + 1]], add=True)
    pltpu.sync_copy(ones_v, deg_sh.at[dst_v.at[b + 1]], add=True)

  pltpu.make_async_copy(y_hbm.at[src_v.at[NB - 1]], bufs[0], gsems[0]).wait()

  plsc.subcore_barrier()

  pltpu.sync_copy(agg_sh.at[pl.ds(r0, RPT)],
                  agg_hbm.at[c].at[pl.ds(r0, RPT)])
  pltpu.sync_copy(deg_sh.at[pl.ds(r0, RPT)],
                  deg_hbm.at[c].at[pl.ds(r0, RPT)])


_seg_deg = pl.kernel(
    _seg_deg_body,
    out_type=(
        jax.ShapeDtypeStruct((NC, N_PAD, DH), jnp.float32),
        jax.ShapeDtypeStruct((NC, N_PAD, DEGW), jnp.float32),
    ),
    mesh=_mesh,
    scratch_types=[
        pltpu.VMEM((NB, K), jnp.int32),
        pltpu.VMEM((NB, K), jnp.int32),
        [pltpu.VMEM((K, DH), jnp.float32)] * 2,
        pltpu.VMEM((ZR, DH), jnp.float32),
        [pltpu.SemaphoreType.DMA] * 2,
        pltpu.VMEM_SHARED((N_PAD, DH), jnp.float32),
        pltpu.VMEM((K, DEGW), jnp.float32),
        pltpu.VMEM((ZR, DEGW), jnp.float32),
        pltpu.VMEM_SHARED((N_PAD, DEGW), jnp.float32),
    ],
    compiler_params=pltpu.CompilerParams(use_tc_tiling_on_sc=False),
)


def _seg_body(src_hbm, dst_hbm, y_hbm, agg_hbm,
              src_v, dst_v, bufs, zbuf, gsems, agg_sh):
  c = lax.axis_index("c")
  s = lax.axis_index("s")
  wid = c * NS + s
  r0 = s * RPT

  @pl.loop(0, ZR)
  def _(r):
    for j in range(DH // 16):
      zbuf[r, pl.ds(j * 16, 16)] = jnp.zeros((16,), jnp.float32)

  @pl.loop(0, 4)
  def _(i):
    pltpu.sync_copy(zbuf, agg_sh.at[pl.ds(r0 + i * ZR, ZR)])

  pltpu.sync_copy(src_hbm.at[wid], src_v)
  pltpu.sync_copy(dst_hbm.at[s], dst_v)

  plsc.subcore_barrier()

  pltpu.async_copy(y_hbm.at[src_v.at[0]], bufs[0], gsems[0])

  @pl.loop(0, NB, step=2)
  def _(b):
    pltpu.async_copy(y_hbm.at[src_v.at[b + 1]], bufs[1], gsems[1])
    pltpu.make_async_copy(y_hbm.at[src_v.at[b]], bufs[0], gsems[0]).wait()
    pltpu.sync_copy(bufs[0], agg_sh.at[dst_v.at[b]], add=True)
    bn = jnp.minimum(b + 2, NB - 1)
    pltpu.async_copy(y_hbm.at[src_v.at[bn]], bufs[0], gsems[0])
    pltpu.make_async_copy(y_hbm.at[src_v.at[b + 1]], bufs[1],
                          gsems[1]).wait()
    pltpu.sync_copy(bufs[1], agg_sh.at[dst_v.at[b + 1]], add=True)

  pltpu.make_async_copy(y_hbm.at[src_v.at[NB - 1]], bufs[0], gsems[0]).wait()

  plsc.subcore_barrier()

  pltpu.sync_copy(agg_sh.at[pl.ds(r0, RPT)],
                  agg_hbm.at[c].at[pl.ds(r0, RPT)])


_seg = pl.kernel(
    _seg_body,
    out_type=jax.ShapeDtypeStruct((NC, N_PAD, DH), jnp.float32),
    mesh=_mesh,
    scratch_types=[
        pltpu.VMEM((NB, K), jnp.int32),
        pltpu.VMEM((NB, K), jnp.int32),
        [pltpu.VMEM((K, DH), jnp.float32)] * 2,
        pltpu.VMEM((ZR, DH), jnp.float32),
        [pltpu.SemaphoreType.DMA] * 2,
        pltpu.VMEM_SHARED((N_PAD, DH), jnp.float32),
    ],
    compiler_params=pltpu.CompilerParams(use_tc_tiling_on_sc=False),
)


RB = 1000


def _mm2_body(x_ref, ws_ref, wn_ref, self_ref, y_ref):
  xb = x_ref[...]
  self_ref[...] = jnp.dot(xb, ws_ref[...], preferred_element_type=jnp.float32)
  y = jnp.dot(xb, wn_ref[...], preferred_element_type=jnp.float32)
  y_ref[0] = y[:, :DH]
  y_ref[1] = y[:, DH:]


_mm2 = pl.pallas_call(
    _mm2_body,
    grid=(N // RB,),
    in_specs=[
        pl.BlockSpec((RB, D), lambda i: (i, 0)),
        pl.BlockSpec((D, D), lambda i: (0, 0)),
        pl.BlockSpec((D, D), lambda i: (0, 0)),
    ],
    out_specs=[
        pl.BlockSpec((RB, D), lambda i: (i, 0)),
        pl.BlockSpec((2, RB, DH), lambda i: (0, i, 0)),
    ],
    out_shape=[
        jax.ShapeDtypeStruct((N, D), jnp.float32),
        jax.ShapeDtypeStruct((2, N, DH), jnp.float32),
    ],
)


def _combine_mm_body(s1_ref, a0_ref, a1_ref, d0_ref, b_ref,
                     ws_ref, wn_ref, self2_ref, y2_ref):
  deg = jnp.maximum(d0_ref[:, :1], 1.0)
  agg = jnp.concatenate([a0_ref[...], a1_ref[...]], axis=1)
  h = jnp.maximum(s1_ref[...] + b_ref[...] + agg / deg, 0.0)
  self2_ref[...] = jnp.dot(h, ws_ref[...], preferred_element_type=jnp.float32)
  y2 = jnp.dot(h, wn_ref[...], preferred_element_type=jnp.float32)
  y2_ref[0] = y2[:, :DH]
  y2_ref[1] = y2[:, DH:]


_combine_mm = pl.pallas_call(
    _combine_mm_body,
    grid=(N // RB,),
    in_specs=[
        pl.BlockSpec((RB, D), lambda i: (i, 0)),
        pl.BlockSpec((RB, DH), lambda i: (i, 0)),
        pl.BlockSpec((RB, DH), lambda i: (i, 0)),
        pl.BlockSpec((RB, DEGW), lambda i: (i, 0)),
        pl.BlockSpec((1, D), lambda i: (0, 0)),
        pl.BlockSpec((D, D), lambda i: (0, 0)),
        pl.BlockSpec((D, D), lambda i: (0, 0)),
    ],
    out_specs=[
        pl.BlockSpec((RB, D), lambda i: (i, 0)),
        pl.BlockSpec((2, RB, DH), lambda i: (0, i, 0)),
    ],
    out_shape=[
        jax.ShapeDtypeStruct((N, D), jnp.float32),
        jax.ShapeDtypeStruct((2, N, DH), jnp.float32),
    ],
)


def _final_body(s2_ref, a0_ref, a1_ref, d0_ref, b_ref, out_ref):
  deg = jnp.maximum(d0_ref[:, :1], 1.0)
  agg = jnp.concatenate([a0_ref[...], a1_ref[...]], axis=1)
  out_ref[...] = s2_ref[...] + b_ref[...] + agg / deg


_final = pl.pallas_call(
    _final_body,
    grid=(N // RB,),
    in_specs=[
        pl.BlockSpec((RB, D), lambda i: (i, 0)),
        pl.BlockSpec((RB, DH), lambda i: (i, 0)),
        pl.BlockSpec((RB, DH), lambda i: (i, 0)),
        pl.BlockSpec((RB, DEGW), lambda i: (i, 0)),
        pl.BlockSpec((1, D), lambda i: (0, 0)),
    ],
    out_specs=pl.BlockSpec((RB, D), lambda i: (i, 0)),
    out_shape=jax.ShapeDtypeStruct((N, D), jnp.float32),
)


@jax.jit
def kernel(edge_index, in_feat, W_self1, W_neigh1, b1, W_self2, W_neigh2, b2):
  src = edge_index[0]
  dst = edge_index[1]
  pad = E_PAD - E
  src_t = jnp.concatenate(
      [src, jnp.zeros((pad,), jnp.int32)]).reshape(NS, NB, K)
  src_r = jnp.concatenate([src_t, src_t + N], axis=0)
  dst_r = jnp.concatenate(
      [dst, jnp.full((pad,), N, jnp.int32)]).reshape(NS, NB, K)

  self1, y1 = _mm2(in_feat, W_self1, W_neigh1)
  agg1, deg = _seg_deg(src_r, dst_r, y1.reshape(2 * N, DH))
  self2, y2 = _combine_mm(self1, agg1[0], agg1[1], deg[0],
                          b1.reshape(1, D), W_self2, W_neigh2)
  agg2 = _seg(src_r, dst_r, y2.reshape(2 * N, DH))
  out = _final(self2, agg2[0], agg2[1], deg[0], b2.reshape(1, D))
  return out


# speedup vs baseline: 1.4403x; 1.3256x over previous
import jax
import jax.numpy as jnp
from jax import lax
from jax.experimental import pallas as pl
from jax.experimental.pallas import tpu as pltpu
from jax.experimental.pallas import tpu_sc as plsc

N = 10000
D = 128
DH = D // 2
E = 320000
NC = 2
NS = 16
K = 128
NB = 158
EPT = NB * K
E_PAD = EPT * NS
RPT = 640
N_PAD = RPT * NS
ZR = 160
DEGW = 16

_mesh = plsc.VectorSubcoreMesh(core_axis_name="c", subcore_axis_name="s")


def _seg_deg_body(src_hbm, dst_hbm, y_hbm, agg_hbm, deg_hbm,
                  src_v, dst_v, bufs, zbuf, gsems, agg_sh,
                  ones_v, zdeg, deg_sh):
  c = lax.axis_index("c")
  s = lax.axis_index("s")
  wid = c * NS + s
  r0 = s * RPT

  @pl.loop(0, ZR)
  def _(r):
    for j in range(DH // 16):
      zbuf[r, pl.ds(j * 16, 16)] = jnp.zeros((16,), jnp.float32)

  @pl.loop(0, 4)
  def _(i):
    pltpu.sync_copy(zbuf, agg_sh.at[pl.ds(r0 + i * ZR, ZR)])

  @pl.loop(0, ZR)
  def _(r):
    zdeg[r, :] = jnp.zeros((DEGW,), jnp.float32)

  @pl.loop(0, K)
  def _(r):
    ones_v[r, :] = jnp.ones((DEGW,), jnp.float32)

  @pl.loop(0, 4)
  def _(i):
    pltpu.sync_copy(zdeg, deg_sh.at[pl.ds(r0 + i * ZR, ZR)])

  pltpu.sync_copy(src_hbm.at[wid], src_v)
  pltpu.sync_copy(dst_hbm.at[s], dst_v)

  plsc.subcore_barrier()

  pltpu.async_copy(y_hbm.at[src_v.at[0]], bufs[0], gsems[0])

  @pl.loop(0, NB, step=2)
  def _(b):
    pltpu.async_copy(y_hbm.at[src_v.at[b + 1]], bufs[1], gsems[1])
    pltpu.make_async_copy(y_hbm.at[src_v.at[b]], bufs[0], gsems[0]).wait()
    pltpu.sync_copy(bufs[0], agg_sh.at[dst_v.at[b]], add=True)
    pltpu.sync_copy(ones_v, deg_sh.at[dst_v.at[b]], add=True)
    bn = jnp.minimum(b + 2, NB - 1)
    pltpu.async_copy(y_hbm.at[src_v.at[bn]], bufs[0], gsems[0])
    pltpu.make_async_copy(y_hbm.at[src_v.at[b + 1]], bufs[1],
                          gsems[1]).wait()
    pltpu.sync_copy(bufs[1], agg_sh.at[dst_v.at[b + 1]], add=True)
    pltpu.sync_copy(ones_v, deg_sh.at[dst_v.at[b + 1]], add=True)

  pltpu.make_async_copy(y_hbm.at[src_v.at[NB - 1]], bufs[0], gsems[0]).wait()

  plsc.subcore_barrier()

  pltpu.sync_copy(agg_sh.at[pl.ds(r0, RPT)],
                  agg_hbm.at[c].at[pl.ds(r0, RPT)])
  pltpu.sync_copy(deg_sh.at[pl.ds(r0, RPT)],
                  deg_hbm.at[c].at[pl.ds(r0, RPT)])


_seg_deg = pl.kernel(
    _seg_deg_body,
    out_type=(
        jax.ShapeDtypeStruct((NC, N_PAD, DH), jnp.float32),
        jax.ShapeDtypeStruct((NC, N_PAD, DEGW), jnp.float32),
    ),
    mesh=_mesh,
    scratch_types=[
        pltpu.VMEM((NB, K), jnp.int32),
        pltpu.VMEM((NB, K), jnp.int32),
        [pltpu.VMEM((K, DH), jnp.float32)] * 2,
        pltpu.VMEM((ZR, DH), jnp.float32),
        [pltpu.SemaphoreType.DMA] * 2,
        pltpu.VMEM_SHARED((N_PAD, DH), jnp.float32),
        pltpu.VMEM((K, DEGW), jnp.float32),
        pltpu.VMEM((ZR, DEGW), jnp.float32),
        pltpu.VMEM_SHARED((N_PAD, DEGW), jnp.float32),
    ],
    compiler_params=pltpu.CompilerParams(use_tc_tiling_on_sc=False),
)


def _seg_body(src_hbm, dst_hbm, y_hbm, agg_hbm,
              src_v, dst_v, bufs, zbuf, gsems, agg_sh):
  c = lax.axis_index("c")
  s = lax.axis_index("s")
  wid = c * NS + s
  r0 = s * RPT

  @pl.loop(0, ZR)
  def _(r):
    for j in range(DH // 16):
      zbuf[r, pl.ds(j * 16, 16)] = jnp.zeros((16,), jnp.float32)

  @pl.loop(0, 4)
  def _(i):
    pltpu.sync_copy(zbuf, agg_sh.at[pl.ds(r0 + i * ZR, ZR)])

  pltpu.sync_copy(src_hbm.at[wid], src_v)
  pltpu.sync_copy(dst_hbm.at[s], dst_v)

  plsc.subcore_barrier()

  pltpu.async_copy(y_hbm.at[src_v.at[0]], bufs[0], gsems[0])

  @pl.loop(0, NB, step=2)
  def _(b):
    pltpu.async_copy(y_hbm.at[src_v.at[b + 1]], bufs[1], gsems[1])
    pltpu.make_async_copy(y_hbm.at[src_v.at[b]], bufs[0], gsems[0]).wait()
    pltpu.sync_copy(bufs[0], agg_sh.at[dst_v.at[b]], add=True)
    bn = jnp.minimum(b + 2, NB - 1)
    pltpu.async_copy(y_hbm.at[src_v.at[bn]], bufs[0], gsems[0])
    pltpu.make_async_copy(y_hbm.at[src_v.at[b + 1]], bufs[1],
                          gsems[1]).wait()
    pltpu.sync_copy(bufs[1], agg_sh.at[dst_v.at[b + 1]], add=True)

  pltpu.make_async_copy(y_hbm.at[src_v.at[NB - 1]], bufs[0], gsems[0]).wait()

  plsc.subcore_barrier()

  pltpu.sync_copy(agg_sh.at[pl.ds(r0, RPT)],
                  agg_hbm.at[c].at[pl.ds(r0, RPT)])


_seg = pl.kernel(
    _seg_body,
    out_type=jax.ShapeDtypeStruct((NC, N_PAD, DH), jnp.float32),
    mesh=_mesh,
    scratch_types=[
        pltpu.VMEM((NB, K), jnp.int32),
        pltpu.VMEM((NB, K), jnp.int32),
        [pltpu.VMEM((K, DH), jnp.float32)] * 2,
        pltpu.VMEM((ZR, DH), jnp.float32),
        [pltpu.SemaphoreType.DMA] * 2,
        pltpu.VMEM_SHARED((N_PAD, DH), jnp.float32),
    ],
    compiler_params=pltpu.CompilerParams(use_tc_tiling_on_sc=False),
)


RB = 1000


def _mm2_body(x_ref, ws_ref, wn_ref, self_ref, y_ref):
  xb = x_ref[...]
  self_ref[...] = jnp.dot(xb, ws_ref[...], preferred_element_type=jnp.float32)
  y = jnp.dot(xb, wn_ref[...], preferred_element_type=jnp.float32)
  y_ref[0] = y[:, :DH]
  y_ref[1] = y[:, DH:]


_mm2 = pl.pallas_call(
    _mm2_body,
    grid=(N // RB,),
    in_specs=[
        pl.BlockSpec((RB, D), lambda i: (i, 0)),
        pl.BlockSpec((D, D), lambda i: (0, 0)),
        pl.BlockSpec((D, D), lambda i: (0, 0)),
    ],
    out_specs=[
        pl.BlockSpec((RB, D), lambda i: (i, 0)),
        pl.BlockSpec((2, RB, DH), lambda i: (0, i, 0)),
    ],
    out_shape=[
        jax.ShapeDtypeStruct((N, D), jnp.float32),
        jax.ShapeDtypeStruct((2, N, DH), jnp.float32),
    ],
)


def _combine_mm_body(s1_ref, a0_ref, a1_ref, d0_ref, b_ref,
                     ws_ref, wn_ref, self2_ref, y2_ref):
  deg = jnp.maximum(d0_ref[:, :1], 1.0)
  agg = jnp.concatenate([a0_ref[...], a1_ref[...]], axis=1)
  h = jnp.maximum(s1_ref[...] + b_ref[...] + agg / deg, 0.0)
  self2_ref[...] = jnp.dot(h, ws_ref[...], preferred_element_type=jnp.float32)
  y2 = jnp.dot(h, wn_ref[...], preferred_element_type=jnp.float32)
  y2_ref[0] = y2[:, :DH]
  y2_ref[1] = y2[:, DH:]


_combine_mm = pl.pallas_call(
    _combine_mm_body,
    grid=(N // RB,),
    in_specs=[
        pl.BlockSpec((RB, D), lambda i: (i, 0)),
        pl.BlockSpec((RB, DH), lambda i: (i, 0)),
        pl.BlockSpec((RB, DH), lambda i: (i, 0)),
        pl.BlockSpec((RB, DEGW), lambda i: (i, 0)),
        pl.BlockSpec((1, D), lambda i: (0, 0)),
        pl.BlockSpec((D, D), lambda i: (0, 0)),
        pl.BlockSpec((D, D), lambda i: (0, 0)),
    ],
    out_specs=[
        pl.BlockSpec((RB, D), lambda i: (i, 0)),
        pl.BlockSpec((2, RB, DH), lambda i: (0, i, 0)),
    ],
    out_shape=[
        jax.ShapeDtypeStruct((N, D), jnp.float32),
        jax.ShapeDtypeStruct((2, N, DH), jnp.float32),
    ],
)


def _final_body(s2_ref, a0_ref, a1_ref, d0_ref, b_ref, out_ref):
  deg = jnp.maximum(d0_ref[:, :1], 1.0)
  agg = jnp.concatenate([a0_ref[...], a1_ref[...]], axis=1)
  out_ref[...] = s2_ref[...] + b_ref[...] + agg / deg


_final = pl.pallas_call(
    _final_body,
    grid=(N // RB,),
    in_specs=[
        pl.BlockSpec((RB, D), lambda i: (i, 0)),
        pl.BlockSpec((RB, DH), lambda i: (i, 0)),
        pl.BlockSpec((RB, DH), lambda i: (i, 0)),
        pl.BlockSpec((RB, DEGW), lambda i: (i, 0)),
        pl.BlockSpec((1, D), lambda i: (0, 0)),
    ],
    out_specs=pl.BlockSpec((RB, D), lambda i: (i, 0)),
    out_shape=jax.ShapeDtypeStruct((N, D), jnp.float32),
)


@jax.jit
def kernel(edge_index, in_feat, W_self1, W_neigh1, b1, W_self2, W_neigh2, b2):
  src = edge_index[0]
  dst = edge_index[1]
  pad = E_PAD - E
  src_t = jnp.concatenate(
      [src, jnp.zeros((pad,), jnp.int32)]).reshape(NS, NB, K)
  src_r = jnp.concatenate([src_t, src_t + N], axis=0)
  pad_dst = N + jnp.arange(pad, dtype=jnp.int32) % (N_PAD - N)
  dst_r = jnp.concatenate([dst, pad_dst]).reshape(NS, NB, K)

  self1, y1 = _mm2(in_feat, W_self1, W_neigh1)
  agg1, deg = _seg_deg(src_r, dst_r, y1.reshape(2 * N, DH))
  self2, y2 = _combine_mm(self1, agg1[0], agg1[1], deg[0],
                          b1.reshape(1, D), W_self2, W_neigh2)
  agg2 = _seg(src_r, dst_r, y2.reshape(2 * N, DH))
  out = _final(self2, agg2[0], agg2[1], deg[0], b2.reshape(1, D))
  return out
